# baseline (speedup 1.0000x reference)
import jax
import jax.numpy as jnp
from jax import lax
from jax.experimental import pallas as pl
from jax.experimental.pallas import tpu as pltpu
from jax.experimental.pallas import tpu_sc as plsc

D = 128
EPS = 1e-12
NC = 2
NS = 16
NW = NC * NS
CHUNK = 128
NVR = D // 16
ROWS = 512


def _rsqrt_newton(x):
    i = plsc.bitcast(x, jnp.int32)
    i = jnp.int32(0x5F3759DF) - lax.shift_right_arithmetic(i, 1)
    y = plsc.bitcast(i, jnp.float32)
    half = jnp.float32(0.5) * x
    for _ in range(2):
        y = y * (jnp.float32(1.5) - half * y * y)
    return y


def _tree_sum(vs):
    while len(vs) > 1:
        vs = [vs[i] + vs[i + 1] for i in range(0, len(vs) - 1, 2)] + (
            [vs[-1]] if len(vs) % 2 else [])
    return vs[0]


def _fused_body(idx_hbm, tid_hbm, table_hbm, ttab_hbm, pos_hbm, gam_hbm,
                bet_hbm, out_hbm, idx_v, tid_v, wbuf0, wbuf1, pbuf0, pbuf1,
                const_v, wsem0, wsem1, psem0, psem1):
    wid = lax.axis_index("s") * NC + lax.axis_index("c")
    batch, seq_len = idx_hbm.shape
    rows_per_w = seq_len * batch // NW
    runs_per_row = seq_len // rows_per_w
    n_chunks = rows_per_w // CHUNK
    rb = wid // runs_per_row
    pos0 = (wid % runs_per_row) * rows_per_w
    base_row = wid * rows_per_w

    for jj in range(n_chunks):
        pltpu.sync_copy(idx_hbm.at[rb, pl.ds(pos0 + jj * CHUNK, CHUNK)],
                        idx_v.at[jj])
        pltpu.sync_copy(tid_hbm.at[rb, pl.ds(pos0 + jj * CHUNK, CHUNK)],
                        tid_v.at[jj])
    pltpu.sync_copy(gam_hbm, const_v.at[0])
    pltpu.sync_copy(bet_hbm, const_v.at[1])
    pltpu.sync_copy(ttab_hbm.at[0], const_v.at[2])
    pltpu.sync_copy(ttab_hbm.at[1], const_v.at[3])

    scale = jnp.float32(float(D) ** 0.5)
    inv_d = jnp.float32(1.0 / D)
    gdn = lax.GatherDimensionNumbers(
        offset_dims=(), collapsed_slice_dims=(0,), start_index_map=(0,))

    gam_c = [const_v[0, pl.ds(r * 16, 16)] for r in range(NVR)]
    bet_c = [const_v[1, pl.ds(r * 16, 16)] for r in range(NVR)]
    ty0_c = [scale * const_v[2, pl.ds(r * 16, 16)] for r in range(NVR)]
    ty1_c = [scale * const_v[3, pl.ds(r * 16, 16)] for r in range(NVR)]

    wbufs = (wbuf0, wbuf1)
    pbufs = (pbuf0, pbuf1)
    wsems = (wsem0, wsem1)
    psems = (psem0, psem1)

    def start(jj, b):
        pltpu.async_copy(table_hbm.at[idx_v.at[jj]], wbufs[b], wsems[b])
        pltpu.async_copy(pos_hbm.at[pl.ds(pos0 + jj * CHUNK, CHUNK)],
                         pbufs[b], psems[b])

    def wait(jj, b):
        pltpu.make_async_copy(table_hbm.at[idx_v.at[jj]], wbufs[b],
                              wsems[b]).wait()
        pltpu.make_async_copy(pos_hbm.at[pl.ds(pos0 + jj * CHUNK, CHUNK)],
                              pbufs[b], psems[b]).wait()

    def process(jj, b):
        wb = wbufs[b]
        pb = pbufs[b]
        tid_row = tid_v.at[jj]

        @plsc.parallel_loop(0, CHUNK, unroll=2)
        def token(i):
            g16 = lax.shift_right_logical(i, 4)
            lane = lax.bitwise_and(i, 15)
            tid16 = tid_row[pl.ds(g16 * 16, 16)]
            lane_v = jnp.full((16, 1), lane, dtype=jnp.int32)
            tid_splat = lax.gather(
                tid16, lane_v, gdn, (1,),
                mode=lax.GatherScatterMode.PROMISE_IN_BOUNDS)
            is0 = tid_splat == 0
            xs = []
            sqs = []
            for r in range(NVR):
                tok = wb[i, pl.ds(r * 16, 16)]
                posr = pb[i, pl.ds(r * 16, 16)]
                te = jnp.where(is0, ty0_c[r], ty1_c[r])
                x = scale * tok + te + posr
                xs.append(x)
                sqs.append(x * x)
            ssum = _tree_sum(list(xs))
            ssq = _tree_sum(sqs)
            tsum = jnp.broadcast_to(lax.reduce_sum(ssum, (0,)), (16,))
            tsq = jnp.broadcast_to(lax.reduce_sum(ssq, (0,)), (16,))
            mean = tsum * inv_d
            var = tsq * inv_d - mean * mean
            rstd = _rsqrt_newton(var + jnp.float32(EPS))
            for r in range(NVR):
                wb[i, pl.ds(r * 16, 16)] = (
                    (xs[r] - mean) * (gam_c[r] * rstd) + bet_c[r])

        pltpu.sync_copy(wb, out_hbm.at[pl.ds(base_row + jj * CHUNK, CHUNK)])

    start(0, 0)

    def two_chunks(k, carry):
        j0 = 2 * k
        wait(j0, 0)
        start(j0 + 1, 1)
        process(j0, 0)
        wait(j0 + 1, 1)

        @pl.when(j0 + 2 < n_chunks)
        def _():
            start(j0 + 2, 0)

        process(j0 + 1, 1)
        return carry

    lax.fori_loop(0, n_chunks // 2, two_chunks, 0)


def _fused_call(idx2, tid2, word_table, type_table, pos_table, gamma, beta):
    batch, seq_len = idx2.shape
    t_rows = batch * seq_len
    n_chunks = t_rows // CHUNK // NW
    fn = pl.kernel(
        _fused_body,
        out_type=jax.ShapeDtypeStruct((t_rows, D), jnp.float32),
        mesh=plsc.VectorSubcoreMesh(core_axis_name="c", subcore_axis_name="s"),
        scratch_types=[
            pltpu.VMEM((n_chunks, CHUNK), jnp.int32),
            pltpu.VMEM((n_chunks, CHUNK), jnp.int32),
            pltpu.VMEM((CHUNK, D), jnp.float32),
            pltpu.VMEM((CHUNK, D), jnp.float32),
            pltpu.VMEM((CHUNK, D), jnp.float32),
            pltpu.VMEM((CHUNK, D), jnp.float32),
            pltpu.VMEM((4, D), jnp.float32),
            pltpu.SemaphoreType.DMA,
            pltpu.SemaphoreType.DMA,
            pltpu.SemaphoreType.DMA,
            pltpu.SemaphoreType.DMA,
        ],
        compiler_params=pltpu.CompilerParams(needs_layout_passes=False),
    )
    return fn(idx2, tid2, word_table, type_table, pos_table, gamma, beta)


def _gather_body(idx_hbm, table_hbm, out_hbm, idx_v, buf0, buf1, sem0, sem1):
    wid = lax.axis_index("s") * NC + lax.axis_index("c")
    batch, seq_len = idx_hbm.shape
    rows_per_w = seq_len * batch // NW
    runs_per_row = seq_len // rows_per_w
    n_chunks = rows_per_w // CHUNK
    rb = wid // runs_per_row
    pos0 = (wid % runs_per_row) * rows_per_w
    base_row = wid * rows_per_w

    for jj in range(n_chunks):
        pltpu.sync_copy(idx_hbm.at[rb, pl.ds(pos0 + jj * CHUNK, CHUNK)],
                        idx_v.at[jj])
    bufs = (buf0, buf1)
    sems = (sem0, sem1)

    def start(j, b):
        return pltpu.async_copy(table_hbm.at[idx_v.at[j]], bufs[b], sems[b])

    cp = start(0, 0)
    for j in range(n_chunks):
        b = j & 1
        nxt = start(j + 1, 1 - b) if j + 1 < n_chunks else None
        cp.wait()
        pltpu.sync_copy(bufs[b],
                        out_hbm.at[pl.ds(base_row + j * CHUNK, CHUNK)])
        cp = nxt


def _gather_call(idx2, table):
    batch, seq_len = idx2.shape
    t_rows = batch * seq_len
    n_chunks = t_rows // CHUNK // NW
    fn = pl.kernel(
        _gather_body,
        out_type=jax.ShapeDtypeStruct((t_rows, D), jnp.float32),
        mesh=plsc.VectorSubcoreMesh(core_axis_name="c", subcore_axis_name="s"),
        scratch_types=[
            pltpu.VMEM((n_chunks, CHUNK), jnp.int32),
            pltpu.VMEM((CHUNK, D), jnp.float32),
            pltpu.VMEM((CHUNK, D), jnp.float32),
            pltpu.SemaphoreType.DMA,
            pltpu.SemaphoreType.DMA,
        ],
    )
    return fn(idx2, table)


def _ln_body(g_ref, tf_ref, tt_ref, pos_ref, gam_ref, bet_ref, o_ref):
    scale = jnp.sqrt(jnp.float32(D))
    g = g_ref[...]
    t = tf_ref[...]
    tt = tt_ref[...]
    te = tt[0:1, :] + t * (tt[1:2, :] - tt[0:1, :])
    x = scale * (g + te) + pos_ref[...]
    ones = jnp.full((D, D), 1.0 / D, dtype=jnp.float32)
    mean = lax.dot_general(x, ones, (((1,), (0,)), ((), ())),
                           preferred_element_type=jnp.float32)
    xc = x - mean
    var = lax.dot_general(xc * xc, ones, (((1,), (0,)), ((), ())),
                          preferred_element_type=jnp.float32)
    o_ref[...] = xc * lax.rsqrt(var + EPS) * gam_ref[...] + bet_ref[...]


def _ln_call(gathered, tf, type_table, pos_table, gamma, beta, seq_len):
    t_rows = gathered.shape[0]
    n_s = seq_len // ROWS
    n_b = t_rows // seq_len
    return pl.pallas_call(
        _ln_body,
        grid=(n_s, n_b),
        in_specs=[
            pl.BlockSpec((ROWS, D), lambda i, j: (j * n_s + i, 0)),
            pl.BlockSpec((ROWS, 1), lambda i, j: (j * n_s + i, 0)),
            pl.BlockSpec((2, D), lambda i, j: (0, 0)),
            pl.BlockSpec((ROWS, D), lambda i, j: (i, 0)),
            pl.BlockSpec((1, D), lambda i, j: (0, 0)),
            pl.BlockSpec((1, D), lambda i, j: (0, 0)),
        ],
        out_specs=pl.BlockSpec((ROWS, D), lambda i, j: (j * n_s + i, 0)),
        out_shape=jax.ShapeDtypeStruct((t_rows, D), jnp.float32),
    )(gathered, tf, type_table, pos_table, gamma, beta)


def kernel(token_ids, type_ids, word_table, type_table, pos_table,
           ln_gamma, ln_beta):
    b, s = token_ids.shape
    ab = b // 2
    tok = token_ids.astype(jnp.int32)
    tid = type_ids.astype(jnp.int32)

    g_a = _gather_call(tok[:ab], word_table)
    out_b = _fused_call(tok[ab:], tid[ab:], word_table, type_table,
                        pos_table, ln_gamma, ln_beta)
    tf_a = type_ids[:ab].astype(jnp.float32).reshape(ab * s, 1)
    out_a = _ln_call(g_a, tf_a, type_table, pos_table,
                     ln_gamma.reshape(1, D), ln_beta.reshape(1, D), s)
    return jnp.concatenate(
        [out_a.reshape(ab, s, D), out_b.reshape(b - ab, s, D)], axis=0)


# speedup vs baseline: 1.2845x; 1.2845x over previous
import jax
import jax.numpy as jnp
from jax import lax
from jax.experimental import pallas as pl
from jax.experimental.pallas import tpu as pltpu
from jax.experimental.pallas import tpu_sc as plsc

D = 128
EPS = 1e-12
NC = 2
NS = 16
NW = NC * NS
CHUNK = 128
NVR = D // 16


def _rsqrt_newton(x):
    i = plsc.bitcast(x, jnp.int32)
    i = jnp.int32(0x5F3759DF) - lax.shift_right_arithmetic(i, 1)
    y = plsc.bitcast(i, jnp.float32)
    half = jnp.float32(0.5) * x
    y = y * (jnp.float32(1.5) - half * y * y)
    return y


def _tree_sum(vs):
    while len(vs) > 1:
        vs = [vs[i] + vs[i + 1] for i in range(0, len(vs) - 1, 2)] + (
            [vs[-1]] if len(vs) % 2 else [])
    return vs[0]


def _fused_body(idx_hbm, tid_hbm, table_hbm, ttab_hbm, pos_hbm, gam_hbm,
                bet_hbm, out_hbm, idx_v, tid_v, wbuf0, wbuf1, pbuf0, pbuf1,
                obuf0, obuf1, const_v, wsem0, wsem1, psem0, psem1,
                osem0, osem1, ssem):
    wid = lax.axis_index("s") * NC + lax.axis_index("c")
    batch, seq_len = idx_hbm.shape
    rows_per_w = seq_len * batch // NW
    runs_per_row = seq_len // rows_per_w
    n_chunks = rows_per_w // CHUNK
    rb = wid // runs_per_row
    pos0 = (wid % runs_per_row) * rows_per_w
    base_row = wid * rows_per_w

    stage = []
    for jj in range(n_chunks):
        stage.append(pltpu.async_copy(
            idx_hbm.at[rb, pl.ds(pos0 + jj * CHUNK, CHUNK)], idx_v.at[jj],
            ssem))
        stage.append(pltpu.async_copy(
            tid_hbm.at[rb, pl.ds(pos0 + jj * CHUNK, CHUNK)], tid_v.at[jj],
            ssem))
    stage.append(pltpu.async_copy(gam_hbm, const_v.at[0], ssem))
    stage.append(pltpu.async_copy(bet_hbm, const_v.at[1], ssem))
    stage.append(pltpu.async_copy(ttab_hbm.at[0], const_v.at[2], ssem))
    stage.append(pltpu.async_copy(ttab_hbm.at[1], const_v.at[3], ssem))
    for cp in stage:
        cp.wait()

    scale = jnp.float32(float(D) ** 0.5)
    inv_d = jnp.float32(1.0 / D)
    gdn = lax.GatherDimensionNumbers(
        offset_dims=(), collapsed_slice_dims=(0,), start_index_map=(0,))

    gam_c = [const_v[0, pl.ds(r * 16, 16)] for r in range(NVR)]
    bet_c = [const_v[1, pl.ds(r * 16, 16)] for r in range(NVR)]
    ty0_c = [scale * const_v[2, pl.ds(r * 16, 16)] for r in range(NVR)]
    ty1_c = [scale * const_v[3, pl.ds(r * 16, 16)] for r in range(NVR)]

    wbufs = (wbuf0, wbuf1)
    pbufs = (pbuf0, pbuf1)
    obufs = (obuf0, obuf1)
    wsems = (wsem0, wsem1)
    psems = (psem0, psem1)
    osems = (osem0, osem1)

    def start(jj, b):
        pltpu.async_copy(table_hbm.at[idx_v.at[jj]], wbufs[b], wsems[b])
        pltpu.async_copy(pos_hbm.at[pl.ds(pos0 + jj * CHUNK, CHUNK)],
                         pbufs[b], psems[b])

    def wait(jj, b):
        pltpu.make_async_copy(table_hbm.at[idx_v.at[jj]], wbufs[b],
                              wsems[b]).wait()
        pltpu.make_async_copy(pos_hbm.at[pl.ds(pos0 + jj * CHUNK, CHUNK)],
                              pbufs[b], psems[b]).wait()

    def out_start(jj, b):
        pltpu.async_copy(obufs[b],
                         out_hbm.at[pl.ds(base_row + jj * CHUNK, CHUNK)],
                         osems[b])

    def out_drain(jj, b):
        pltpu.make_async_copy(obufs[b],
                              out_hbm.at[pl.ds(base_row + jj * CHUNK, CHUNK)],
                              osems[b]).wait()

    def process(jj, b):
        wb = wbufs[b]
        pb = pbufs[b]
        ob = obufs[b]
        tid_row = tid_v.at[jj]

        @plsc.parallel_loop(0, CHUNK, unroll=2)
        def token(i):
            g16 = lax.shift_right_logical(i, 4)
            lane = lax.bitwise_and(i, 15)
            tid16 = tid_row[pl.ds(g16 * 16, 16)]
            lane_v = jnp.full((16, 1), lane, dtype=jnp.int32)
            tid_splat = lax.gather(
                tid16, lane_v, gdn, (1,),
                mode=lax.GatherScatterMode.PROMISE_IN_BOUNDS)
            is0 = tid_splat == 0
            xs = []
            sqs = []
            for r in range(NVR):
                tok = wb[i, pl.ds(r * 16, 16)]
                posr = pb[i, pl.ds(r * 16, 16)]
                te = jnp.where(is0, ty0_c[r], ty1_c[r])
                x = scale * tok + te + posr
                xs.append(x)
                sqs.append(x * x)
            ssum = _tree_sum(list(xs))
            ssq = _tree_sum(sqs)
            tsum = jnp.broadcast_to(lax.reduce_sum(ssum, (0,)), (16,))
            tsq = jnp.broadcast_to(lax.reduce_sum(ssq, (0,)), (16,))
            mean = tsum * inv_d
            var = tsq * inv_d - mean * mean
            rstd = _rsqrt_newton(var + jnp.float32(EPS))
            for r in range(NVR):
                ob[i, pl.ds(r * 16, 16)] = (
                    (xs[r] - mean) * (gam_c[r] * rstd) + bet_c[r])

    start(0, 0)
    start(1, 1)

    def two_chunks(k, carry):
        j0 = 2 * k
        wait(j0, 0)

        @pl.when(j0 >= 2)
        def _():
            out_drain(j0 - 2, 0)

        process(j0, 0)
        out_start(j0, 0)

        @pl.when(j0 + 2 < n_chunks)
        def _():
            start(j0 + 2, 0)

        wait(j0 + 1, 1)

        @pl.when(j0 >= 1)
        def _():
            out_drain(j0 - 1, 1)

        process(j0 + 1, 1)
        out_start(j0 + 1, 1)

        @pl.when(j0 + 3 < n_chunks)
        def _():
            start(j0 + 3, 1)

        return carry

    lax.fori_loop(0, n_chunks // 2, two_chunks, 0)
    out_drain(n_chunks - 2, 0)
    out_drain(n_chunks - 1, 1)


def kernel(token_ids, type_ids, word_table, type_table, pos_table,
           ln_gamma, ln_beta):
    b, s = token_ids.shape
    t_rows = b * s
    n_chunks = t_rows // CHUNK // NW
    fn = pl.kernel(
        _fused_body,
        out_type=jax.ShapeDtypeStruct((t_rows, D), jnp.float32),
        mesh=plsc.VectorSubcoreMesh(core_axis_name="c", subcore_axis_name="s"),
        scratch_types=[
            pltpu.VMEM((n_chunks, CHUNK), jnp.int32),
            pltpu.VMEM((n_chunks, CHUNK), jnp.int32),
            pltpu.VMEM((CHUNK, D), jnp.float32),
            pltpu.VMEM((CHUNK, D), jnp.float32),
            pltpu.VMEM((CHUNK, D), jnp.float32),
            pltpu.VMEM((CHUNK, D), jnp.float32),
            pltpu.VMEM((CHUNK, D), jnp.float32),
            pltpu.VMEM((CHUNK, D), jnp.float32),
            pltpu.VMEM((4, D), jnp.float32),
            pltpu.SemaphoreType.DMA,
            pltpu.SemaphoreType.DMA,
            pltpu.SemaphoreType.DMA,
            pltpu.SemaphoreType.DMA,
            pltpu.SemaphoreType.DMA,
            pltpu.SemaphoreType.DMA,
            pltpu.SemaphoreType.DMA,
        ],
        compiler_params=pltpu.CompilerParams(
            needs_layout_passes=False, disable_bounds_checks=True,
            disable_semaphore_checks=True),
    )
    out2d = fn(token_ids.astype(jnp.int32), type_ids.astype(jnp.int32),
               word_table, type_table, pos_table, ln_gamma, ln_beta)
    return out2d.reshape(b, s, D)
